# baseline (speedup 1.0000x reference)
import functools

import jax
import jax.numpy as jnp
from jax import lax
from jax.experimental import pallas as pl
from jax.experimental.pallas import tpu as pltpu
from jax.experimental.pallas import tpu_sc as plsc

N_USERS = 100000
N_ITEMS = 100000
N_FEATS = 128
BATCH = 16384

NUM_CORES = 2
NUM_SUBCORES = 16
NUM_WORKERS = NUM_CORES * NUM_SUBCORES
PER_WORKER = BATCH // NUM_WORKERS
CHUNK = 128
NUM_CHUNKS = PER_WORKER // CHUNK
GROUPS = CHUNK // 16


def _body(user_hbm, item_hbm, neg_hbm, ibias_hbm, uemb_hbm, iemb_hbm,
          out_hbm,
          uidx, iidx, nidx, urows, irows, nrows, ibv, nibv, outv, pscr, sem):
  wid = lax.axis_index("s") * NUM_CORES + lax.axis_index("c")
  base = wid * PER_WORKER
  lane = lax.iota(jnp.int32, 16)
  row_idx = [lane + g * 16 for g in range(GROUPS)]

  for c in range(NUM_CHUNKS):
    cbase = base + c * CHUNK
    pltpu.sync_copy(user_hbm.at[pl.ds(cbase, CHUNK)], uidx)
    pltpu.sync_copy(item_hbm.at[pl.ds(cbase, CHUNK)], iidx)
    pltpu.sync_copy(neg_hbm.at[pl.ds(cbase, CHUNK)], nidx)
    cps = [
        pltpu.async_copy(uemb_hbm.at[uidx], urows, sem),
        pltpu.async_copy(iemb_hbm.at[iidx], irows, sem),
        pltpu.async_copy(iemb_hbm.at[nidx], nrows, sem),
        pltpu.async_copy(ibias_hbm.at[iidx], ibv, sem),
        pltpu.async_copy(ibias_hbm.at[nidx], nibv, sem),
    ]
    for cp in cps:
      cp.wait()

    @plsc.parallel_loop(0, CHUNK, unroll=2)
    def _row(r):
      ts = []
      for s in range(N_FEATS // 16):
        sl = pl.ds(s * 16, 16)
        ts.append(urows[r, sl] * (irows[r, sl] - nrows[r, sl]))
      t01, t23 = ts[0] + ts[1], ts[2] + ts[3]
      t45, t67 = ts[4] + ts[5], ts[6] + ts[7]
      pscr[r, pl.ds(0, 16)] = (t01 + t23) + (t45 + t67)

    for g in range(GROUPS):
      acc = ibv[pl.ds(g * 16, 16)] - nibv[pl.ds(g * 16, 16)]
      for l in range(16):
        col = jnp.full((16,), l, dtype=jnp.int32)
        acc = acc + plsc.load_gather(pscr, [row_idx[g], col])
      outv[pl.ds(g * 16, 16)] = acc
    pltpu.sync_copy(outv, out_hbm.at[pl.ds(cbase, CHUNK)])


@jax.jit
def _run(user, item, negative, i_bias_flat, u_embed_w, i_embed_w):
  mesh = plsc.VectorSubcoreMesh(core_axis_name="c", subcore_axis_name="s")
  kfn = functools.partial(
      pl.kernel,
      mesh=mesh,
      compiler_params=pltpu.CompilerParams(needs_layout_passes=False),
      out_type=jax.ShapeDtypeStruct((BATCH,), jnp.float32),
      scratch_types=[
          pltpu.VMEM((CHUNK,), jnp.int32),
          pltpu.VMEM((CHUNK,), jnp.int32),
          pltpu.VMEM((CHUNK,), jnp.int32),
          pltpu.VMEM((CHUNK, N_FEATS), jnp.float32),
          pltpu.VMEM((CHUNK, N_FEATS), jnp.float32),
          pltpu.VMEM((CHUNK, N_FEATS), jnp.float32),
          pltpu.VMEM((CHUNK,), jnp.float32),
          pltpu.VMEM((CHUNK,), jnp.float32),
          pltpu.VMEM((CHUNK,), jnp.float32),
          pltpu.VMEM((CHUNK, 17), jnp.float32),
          pltpu.SemaphoreType.DMA,
      ],
  )(_body)
  return kfn(user, item, negative, i_bias_flat, u_embed_w, i_embed_w)


def kernel(user, item, negative, u_bias_w, i_bias_w, u_embed_w, i_embed_w):
  del u_bias_w
  return _run(user.astype(jnp.int32), item.astype(jnp.int32),
              negative.astype(jnp.int32), i_bias_w.reshape(-1),
              u_embed_w, i_embed_w)


# speedup vs baseline: 1.2300x; 1.2300x over previous
import functools

import jax
import jax.numpy as jnp
from jax import lax
from jax.experimental import pallas as pl
from jax.experimental.pallas import tpu as pltpu
from jax.experimental.pallas import tpu_sc as plsc

N_USERS = 100000
N_ITEMS = 100000
N_FEATS = 128
BATCH = 16384

NUM_CORES = 2
NUM_SUBCORES = 16
NUM_WORKERS = NUM_CORES * NUM_SUBCORES
PER_WORKER = BATCH // NUM_WORKERS
CHUNK = 128
NUM_CHUNKS = PER_WORKER // CHUNK
GROUPS = CHUNK // 16


def _body(user_hbm, item_hbm, neg_hbm, ibias_hbm, uemb_hbm, iemb_hbm,
          out_hbm,
          uidx0, iidx0, nidx0, urows0, irows0, nrows0, ibv0, nibv0,
          uidx1, iidx1, nidx1, urows1, irows1, nrows1, ibv1, nibv1,
          outv, pscr, sem0, sem1):
  wid = lax.axis_index("s") * NUM_CORES + lax.axis_index("c")
  base = wid * PER_WORKER
  lane = lax.iota(jnp.int32, 16)
  row_idx = [lane + g * 16 for g in range(GROUPS)]

  bufs = [(uidx0, iidx0, nidx0, urows0, irows0, nrows0, ibv0, nibv0, sem0),
          (uidx1, iidx1, nidx1, urows1, irows1, nrows1, ibv1, nibv1, sem1)]

  def fire(c):
    uidx, iidx, nidx, urows, irows, nrows, ibv, nibv, sem = bufs[c % 2]
    cbase = base + c * CHUNK
    pltpu.sync_copy(user_hbm.at[pl.ds(cbase, CHUNK)], uidx)
    pltpu.sync_copy(item_hbm.at[pl.ds(cbase, CHUNK)], iidx)
    pltpu.sync_copy(neg_hbm.at[pl.ds(cbase, CHUNK)], nidx)
    return [
        pltpu.async_copy(uemb_hbm.at[uidx], urows, sem),
        pltpu.async_copy(iemb_hbm.at[iidx], irows, sem),
        pltpu.async_copy(iemb_hbm.at[nidx], nrows, sem),
        pltpu.async_copy(ibias_hbm.at[iidx], ibv, sem),
        pltpu.async_copy(ibias_hbm.at[nidx], nibv, sem),
    ]

  pending = {0: fire(0)}
  for c in range(NUM_CHUNKS):
    if c + 1 < NUM_CHUNKS:
      pending[c + 1] = fire(c + 1)
    for cp in pending.pop(c):
      cp.wait()
    _, _, _, urows, irows, nrows, ibv, nibv, _ = bufs[c % 2]
    cbase = base + c * CHUNK

    @plsc.parallel_loop(0, CHUNK, unroll=2)
    def _row(r):
      ts = []
      for s in range(N_FEATS // 16):
        sl = pl.ds(s * 16, 16)
        ts.append(urows[r, sl] * (irows[r, sl] - nrows[r, sl]))
      t01, t23 = ts[0] + ts[1], ts[2] + ts[3]
      t45, t67 = ts[4] + ts[5], ts[6] + ts[7]
      pscr[r, pl.ds(0, 16)] = (t01 + t23) + (t45 + t67)

    for g in range(GROUPS):
      acc = ibv[pl.ds(g * 16, 16)] - nibv[pl.ds(g * 16, 16)]
      for l in range(16):
        col = jnp.full((16,), l, dtype=jnp.int32)
        acc = acc + plsc.load_gather(pscr, [row_idx[g], col])
      outv[pl.ds(g * 16, 16)] = acc
    pltpu.sync_copy(outv, out_hbm.at[pl.ds(cbase, CHUNK)])


@jax.jit
def _run(user, item, negative, i_bias_flat, u_embed_w, i_embed_w):
  mesh = plsc.VectorSubcoreMesh(core_axis_name="c", subcore_axis_name="s")
  kfn = functools.partial(
      pl.kernel,
      mesh=mesh,
      compiler_params=pltpu.CompilerParams(needs_layout_passes=False),
      out_type=jax.ShapeDtypeStruct((BATCH,), jnp.float32),
      scratch_types=(
          [pltpu.VMEM((CHUNK,), jnp.int32)] * 3
          + [pltpu.VMEM((CHUNK, N_FEATS), jnp.float32)] * 3
          + [pltpu.VMEM((CHUNK,), jnp.float32)] * 2
      ) * 2 + [
          pltpu.VMEM((CHUNK,), jnp.float32),
          pltpu.VMEM((CHUNK, 17), jnp.float32),
          pltpu.SemaphoreType.DMA,
          pltpu.SemaphoreType.DMA,
      ],
  )(_body)
  return kfn(user, item, negative, i_bias_flat, u_embed_w, i_embed_w)


def kernel(user, item, negative, u_bias_w, i_bias_w, u_embed_w, i_embed_w):
  del u_bias_w
  return _run(user.astype(jnp.int32), item.astype(jnp.int32),
              negative.astype(jnp.int32), i_bias_w.reshape(-1),
              u_embed_w, i_embed_w)


# speedup vs baseline: 1.2947x; 1.0527x over previous
import functools

import jax
import jax.numpy as jnp
from jax import lax
from jax.experimental import pallas as pl
from jax.experimental.pallas import tpu as pltpu
from jax.experimental.pallas import tpu_sc as plsc

N_USERS = 100000
N_ITEMS = 100000
N_FEATS = 128
BATCH = 16384

NUM_CORES = 2
NUM_SUBCORES = 16
NUM_WORKERS = NUM_CORES * NUM_SUBCORES
PER_WORKER = BATCH // NUM_WORKERS
CHUNK = 128
NUM_CHUNKS = PER_WORKER // CHUNK
GROUPS = CHUNK // 16


def _body(user_hbm, item_hbm, neg_hbm, ibias_hbm, uemb_hbm, iemb_hbm,
          out_hbm,
          uidx_all, iidx_all, nidx_all,
          urows0, irows0, nrows0, ibv0, nibv0,
          urows1, irows1, nrows1, ibv1, nibv1,
          outv, pscr, sem0, sem1):
  wid = lax.axis_index("s") * NUM_CORES + lax.axis_index("c")
  base = wid * PER_WORKER
  lane = lax.iota(jnp.int32, 16)
  row_idx = [lane + g * 16 for g in range(GROUPS)]

  idx_cps = [
      pltpu.async_copy(user_hbm.at[pl.ds(base, PER_WORKER)], uidx_all, sem0),
      pltpu.async_copy(item_hbm.at[pl.ds(base, PER_WORKER)], iidx_all, sem0),
      pltpu.async_copy(neg_hbm.at[pl.ds(base, PER_WORKER)], nidx_all, sem0),
  ]
  for cp in idx_cps:
    cp.wait()

  bufs = [(urows0, irows0, nrows0, ibv0, nibv0, sem0),
          (urows1, irows1, nrows1, ibv1, nibv1, sem1)]

  def fire(c):
    urows, irows, nrows, ibv, nibv, sem = bufs[c % 2]
    csl = pl.ds(c * CHUNK, CHUNK)
    uidx, iidx, nidx = uidx_all.at[csl], iidx_all.at[csl], nidx_all.at[csl]
    return [
        pltpu.async_copy(uemb_hbm.at[uidx], urows, sem),
        pltpu.async_copy(iemb_hbm.at[iidx], irows, sem),
        pltpu.async_copy(iemb_hbm.at[nidx], nrows, sem),
        pltpu.async_copy(ibias_hbm.at[iidx], ibv, sem),
        pltpu.async_copy(ibias_hbm.at[nidx], nibv, sem),
    ]

  pending = {0: fire(0)}
  for c in range(NUM_CHUNKS):
    if c + 1 < NUM_CHUNKS:
      pending[c + 1] = fire(c + 1)
    for cp in pending.pop(c):
      cp.wait()
    urows, irows, nrows, ibv, nibv, _ = bufs[c % 2]

    @plsc.parallel_loop(0, CHUNK, unroll=2)
    def _row(r):
      ts = []
      for s in range(N_FEATS // 16):
        sl = pl.ds(s * 16, 16)
        ts.append(urows[r, sl] * (irows[r, sl] - nrows[r, sl]))
      t01, t23 = ts[0] + ts[1], ts[2] + ts[3]
      t45, t67 = ts[4] + ts[5], ts[6] + ts[7]
      pscr[r, pl.ds(0, 16)] = (t01 + t23) + (t45 + t67)

    for g in range(GROUPS):
      acc = ibv[pl.ds(g * 16, 16)] - nibv[pl.ds(g * 16, 16)]
      for l in range(16):
        col = jnp.full((16,), l, dtype=jnp.int32)
        acc = acc + plsc.load_gather(pscr, [row_idx[g], col])
      outv[pl.ds(c * CHUNK + g * 16, 16)] = acc
  pltpu.sync_copy(outv, out_hbm.at[pl.ds(base, PER_WORKER)])


@jax.jit
def _run(user, item, negative, i_bias_flat, u_embed_w, i_embed_w):
  mesh = plsc.VectorSubcoreMesh(core_axis_name="c", subcore_axis_name="s")
  kfn = functools.partial(
      pl.kernel,
      mesh=mesh,
      compiler_params=pltpu.CompilerParams(needs_layout_passes=False),
      out_type=jax.ShapeDtypeStruct((BATCH,), jnp.float32),
      scratch_types=[pltpu.VMEM((PER_WORKER,), jnp.int32)] * 3 + (
          [pltpu.VMEM((CHUNK, N_FEATS), jnp.float32)] * 3
          + [pltpu.VMEM((CHUNK,), jnp.float32)] * 2
      ) * 2 + [
          pltpu.VMEM((PER_WORKER,), jnp.float32),
          pltpu.VMEM((CHUNK, 17), jnp.float32),
          pltpu.SemaphoreType.DMA,
          pltpu.SemaphoreType.DMA,
      ],
  )(_body)
  return kfn(user, item, negative, i_bias_flat, u_embed_w, i_embed_w)


def kernel(user, item, negative, u_bias_w, i_bias_w, u_embed_w, i_embed_w):
  del u_bias_w
  return _run(user.astype(jnp.int32), item.astype(jnp.int32),
              negative.astype(jnp.int32), i_bias_w.reshape(-1),
              u_embed_w, i_embed_w)


# speedup vs baseline: 1.2979x; 1.0024x over previous
import functools

import jax
import jax.numpy as jnp
from jax import lax
from jax.experimental import pallas as pl
from jax.experimental.pallas import tpu as pltpu
from jax.experimental.pallas import tpu_sc as plsc

N_USERS = 100000
N_ITEMS = 100000
N_FEATS = 128
BATCH = 16384

NUM_CORES = 2
NUM_SUBCORES = 16
NUM_WORKERS = NUM_CORES * NUM_SUBCORES
PER_WORKER = BATCH // NUM_WORKERS
CHUNK = 128
NUM_CHUNKS = PER_WORKER // CHUNK
GROUPS = CHUNK // 16


def _body(user_hbm, item_hbm, neg_hbm, ibias_hbm, uemb_hbm, iemb_hbm,
          out_hbm,
          uidx_all, iidx_all, nidx_all, ibv_all, nibv_all,
          urows0, irows0, nrows0,
          urows1, irows1, nrows1,
          outv, pscr, sem0, sem1, sem2):
  wid = lax.axis_index("s") * NUM_CORES + lax.axis_index("c")
  base = wid * PER_WORKER
  lane = lax.iota(jnp.int32, 16)
  row_idx = [lane + g * 16 for g in range(GROUPS)]

  idx_cps = [
      pltpu.async_copy(user_hbm.at[pl.ds(base, PER_WORKER)], uidx_all, sem0),
      pltpu.async_copy(item_hbm.at[pl.ds(base, PER_WORKER)], iidx_all, sem0),
      pltpu.async_copy(neg_hbm.at[pl.ds(base, PER_WORKER)], nidx_all, sem0),
  ]
  for cp in idx_cps:
    cp.wait()

  bias_cps = [
      pltpu.async_copy(ibias_hbm.at[iidx_all], ibv_all, sem2),
      pltpu.async_copy(ibias_hbm.at[nidx_all], nibv_all, sem2),
  ]

  bufs = [(urows0, irows0, nrows0, sem0),
          (urows1, irows1, nrows1, sem1)]

  def fire(c):
    urows, irows, nrows, sem = bufs[c % 2]
    csl = pl.ds(c * CHUNK, CHUNK)
    uidx, iidx, nidx = uidx_all.at[csl], iidx_all.at[csl], nidx_all.at[csl]
    return [
        pltpu.async_copy(uemb_hbm.at[uidx], urows, sem),
        pltpu.async_copy(iemb_hbm.at[iidx], irows, sem),
        pltpu.async_copy(iemb_hbm.at[nidx], nrows, sem),
    ]

  pending = {0: fire(0)}
  for c in range(NUM_CHUNKS):
    if c + 1 < NUM_CHUNKS:
      pending[c + 1] = fire(c + 1)
    for cp in pending.pop(c):
      cp.wait()
    if c == 0:
      for cp in bias_cps:
        cp.wait()
    urows, irows, nrows, _ = bufs[c % 2]

    @plsc.parallel_loop(0, CHUNK, unroll=2)
    def _row(r):
      ts = []
      for s in range(N_FEATS // 16):
        sl = pl.ds(s * 16, 16)
        ts.append(urows[r, sl] * (irows[r, sl] - nrows[r, sl]))
      t01, t23 = ts[0] + ts[1], ts[2] + ts[3]
      t45, t67 = ts[4] + ts[5], ts[6] + ts[7]
      pscr[r, pl.ds(0, 16)] = (t01 + t23) + (t45 + t67)

    for g in range(GROUPS):
      bsl = pl.ds(c * CHUNK + g * 16, 16)
      acc = ibv_all[bsl] - nibv_all[bsl]
      for l in range(16):
        col = jnp.full((16,), l, dtype=jnp.int32)
        acc = acc + plsc.load_gather(pscr, [row_idx[g], col])
      outv[pl.ds(c * CHUNK + g * 16, 16)] = acc
  pltpu.sync_copy(outv, out_hbm.at[pl.ds(base, PER_WORKER)])


@jax.jit
def _run(user, item, negative, i_bias_flat, u_embed_w, i_embed_w):
  mesh = plsc.VectorSubcoreMesh(core_axis_name="c", subcore_axis_name="s")
  kfn = functools.partial(
      pl.kernel,
      mesh=mesh,
      compiler_params=pltpu.CompilerParams(needs_layout_passes=False),
      out_type=jax.ShapeDtypeStruct((BATCH,), jnp.float32),
      scratch_types=[pltpu.VMEM((PER_WORKER,), jnp.int32)] * 3
      + [pltpu.VMEM((PER_WORKER,), jnp.float32)] * 2
      + [pltpu.VMEM((CHUNK, N_FEATS), jnp.float32)] * 6
      + [
          pltpu.VMEM((PER_WORKER,), jnp.float32),
          pltpu.VMEM((CHUNK, 17), jnp.float32),
          pltpu.SemaphoreType.DMA,
          pltpu.SemaphoreType.DMA,
          pltpu.SemaphoreType.DMA,
      ],
  )(_body)
  return kfn(user, item, negative, i_bias_flat, u_embed_w, i_embed_w)


def kernel(user, item, negative, u_bias_w, i_bias_w, u_embed_w, i_embed_w):
  del u_bias_w
  return _run(user.astype(jnp.int32), item.astype(jnp.int32),
              negative.astype(jnp.int32), i_bias_w.reshape(-1),
              u_embed_w, i_embed_w)
